# baseline (speedup 1.0000x reference)
import functools

import jax
import jax.numpy as jnp
from jax import lax
from jax.experimental import pallas as pl
from jax.experimental.pallas import tpu as pltpu
from jax.experimental.pallas import tpu_sc as plsc

VOCAB = 8192
D = 8192
NC, NS = 2, 16
NW = NC * NS
BT = 32 * 512
VPW = VOCAB // NW
NVREG = BT // 16
WR = 4
NWIN = VPW // WR
PAD = 16


def _sc_lookup(table, idx2):
  mesh = plsc.VectorSubcoreMesh(
      core_axis_name="c", subcore_axis_name="s", num_cores=NC,
      num_subcores=NS)

  @functools.partial(
      pl.kernel,
      mesh=mesh,
      compiler_params=pltpu.CompilerParams(needs_layout_passes=False),
      out_type=jax.ShapeDtypeStruct((BT, D), jnp.float32),
      scratch_types=[
          pltpu.VMEM((BT,), jnp.int32),
          pltpu.VMEM((VPW + PAD,), jnp.int32),
          pltpu.VMEM((BT + PAD,), jnp.int32),
          pltpu.VMEM((WR, D), jnp.float32),
          pltpu.VMEM((WR, D), jnp.float32),
          pltpu.SemaphoreType.DMA,
          pltpu.SemaphoreType.DMA,
          pltpu.SemaphoreType.DMA,
          pltpu.SemaphoreType.DMA,
      ],
  )
  def k(table_hbm, idx_hbm, out_hbm, idx_v, nxt_v,
        pos_v, buf0, buf1, gs0, gs1, ss0, ss1):
    cid = lax.axis_index("c")
    sid = lax.axis_index("s")
    wid = sid * NC + cid
    v0 = wid * VPW

    bufs = (buf0, buf1)
    gsems = (gs0, gs1)
    ssems = (ss0, ss1)

    def g_copy(w, b):
      return pltpu.make_async_copy(
          table_hbm.at[pl.ds(v0 + w * WR, WR)], bufs[b], gsems[b])

    g_copy(0, 0).start()
    g_copy(1, 1).start()

    pltpu.sync_copy(idx_hbm, idx_v)

    zeros = jnp.zeros((16,), jnp.int32)
    for j in range(VPW // 16):
      nxt_v[pl.ds(j * 16, 16)] = zeros

    @pl.loop(0, NVREG)
    def _(i):
      x = idx_v[pl.ds(i * 16, 16)]
      m = (x >> 8) == wid
      vl = x & (VPW - 1)
      cnt1, lastm = plsc.scan_count(x, m)
      plsc.addupdate_scatter(nxt_v, [vl], cnt1, mask=m & lastm)

    running = jnp.int32(0)
    for j in range(VPW // 16):
      c = nxt_v[pl.ds(j * 16, 16)]
      s = plsc.cumsum(c)
      nxt_v[pl.ds(j * 16, 16)] = s - c + running
      running = running + jnp.sum(c)

    @pl.loop(0, NVREG)
    def _(i):
      x = idx_v[pl.ds(i * 16, 16)]
      m = (x >> 8) == wid
      vl = x & (VPW - 1)
      cnt1, lastm = plsc.scan_count(x, m)
      base = plsc.load_gather(nxt_v, [vl], mask=m)
      slot = base + cnt1 - 1
      p = i * 16 + lax.iota(jnp.int32, 16)
      plsc.store_scatter(pos_v, [slot], p, mask=m)
      plsc.addupdate_scatter(nxt_v, [vl], cnt1, mask=m & lastm)

    def sload(ref, i):
      return ref[pl.ds(i, 16)][0]

    def drain(b, n):
      @pl.loop(0, n)
      def _(_):
        pltpu.make_async_copy(bufs[b].at[0], out_hbm.at[0], ssems[b]).wait()

    def w_body(w0, carry):
      pend = list(carry)
      for b in range(2):
        w = w0 + b
        ob = 1 - b
        g_copy(w, b).wait()

        n_w = jnp.int32(0)
        for r in range(WR):
          vloc = w * WR + r
          s_raw = sload(nxt_v, jnp.maximum(vloc - 1, 0))
          s_ = jnp.where(vloc < 1, jnp.int32(0), s_raw)
          e_ = sload(nxt_v, vloc)

          @pl.loop(s_, e_)
          def _(mi):
            p = sload(pos_v, mi)
            pltpu.make_async_copy(
                bufs[b].at[r], out_hbm.at[p], ssems[b]).start()

          n_w = n_w + (e_ - s_)

        drain(ob, pend[ob])

        @pl.when((w >= 1) & (w + 1 < NWIN))
        def _():
          g_copy(w + 1, ob).start()

        pend[ob] = jnp.int32(0)
        pend[b] = n_w
      return tuple(pend)

    final = pl.loop(
        0, NWIN, step=2, init_carry=(jnp.int32(0), jnp.int32(0)))(w_body)
    drain(0, final[0])
    drain(1, final[1])

  return k(table, idx2)


def kernel(X, table):
  idx2 = X.reshape(BT)
  out = _sc_lookup(table, idx2)
  return out.reshape(X.shape[0], X.shape[1], VOCAB)


# speedup vs baseline: 1.0128x; 1.0128x over previous
import functools

import jax
import jax.numpy as jnp
from jax import lax
from jax.experimental import pallas as pl
from jax.experimental.pallas import tpu as pltpu
from jax.experimental.pallas import tpu_sc as plsc

VOCAB = 8192
D = 8192
NC, NS = 2, 16
NW = NC * NS
BT = 32 * 512
RPW = BT // NW
K = 4
NCHUNK = RPW // K
NBUF = 2


def _sc_gather(table, idx3):
  mesh = plsc.VectorSubcoreMesh(
      core_axis_name="c", subcore_axis_name="s", num_cores=NC,
      num_subcores=NS)

  @functools.partial(
      pl.kernel,
      mesh=mesh,
      out_type=jax.ShapeDtypeStruct((BT, D), jnp.float32),
      scratch_types=(
          [pltpu.VMEM((NCHUNK, K), jnp.int32)]
          + [pltpu.VMEM((K, D), jnp.float32) for _ in range(NBUF)]
          + [pltpu.SemaphoreType.DMA for _ in range(2 * NBUF)]
      ),
  )
  def k(table_hbm, idx_hbm, out_hbm, idx_v, *bufs_and_sems):
    bufs = bufs_and_sems[:NBUF]
    gsems = bufs_and_sems[NBUF:2 * NBUF]
    ssems = bufs_and_sems[2 * NBUF:]
    cid = lax.axis_index("c")
    sid = lax.axis_index("s")
    wid = sid * NC + cid
    base = wid * RPW

    pltpu.sync_copy(idx_hbm.at[wid], idx_v)

    def g_copy(c, b):
      return pltpu.make_async_copy(
          table_hbm.at[idx_v.at[c]], bufs[b], gsems[b])

    def s_copy(c, b):
      return pltpu.make_async_copy(
          bufs[b], out_hbm.at[pl.ds(base + c * K, K)], ssems[b])

    g_copy(0, 0).start()

    @pl.loop(0, NCHUNK, step=NBUF)
    def _(j0):
      for b in range(NBUF):
        c = j0 + b
        nb = (b + 1) % NBUF
        g_copy(c, b).wait()

        @pl.when(c + 1 >= NBUF)
        def _():
          s_copy(c + 1 - NBUF, nb).wait()

        @pl.when(c + 1 < NCHUNK)
        def _():
          g_copy(c + 1, nb).start()

        s_copy(c, b).start()

    for t in range(NCHUNK - NBUF + 1, NCHUNK):
      s_copy(t, t % NBUF).wait()

  return k(table, idx3)


def kernel(X, table):
  idx3 = X.reshape(NW, NCHUNK, K)
  out = _sc_gather(table, idx3)
  return out.reshape(X.shape[0], X.shape[1], VOCAB)
